# baseline (speedup 1.0000x reference)
import jax
import jax.numpy as jnp
from jax import lax
from jax.experimental import pallas as pl
from jax.experimental.pallas import tpu as pltpu
from jax.experimental.pallas import tpu_sc as plsc

N_NODES = 10000
D_IN = 128
D_HID = 128
D_OUT = 16

N_PAD = 10240
D_AUG = D_HID + 16
NC, NS = 2, 16
NT = NC * NS
CHUNK = 80
GRP = 8
TOT_GRPS = 512
G_SC0 = 16
G_SC1 = TOT_GRPS // NS - G_SC0
ROWS_PER_TILE = N_PAD // NS


def _mm2_body(x_ref, w1_ref, w2_ref, aug_ref, o_ref):
    h = jnp.maximum(jnp.dot(x_ref[...], w1_ref[...],
                            preferred_element_type=jnp.float32), 0.0)
    feat = jnp.dot(h, w2_ref[...], preferred_element_type=jnp.float32)
    o_ref[...] = jnp.concatenate([feat, aug_ref[...]], axis=1)


def _input_matmul(x_pad, W_in, W_conv, aug):
    blk = 1024
    return pl.pallas_call(
        _mm2_body,
        grid=(N_PAD // blk,),
        in_specs=[
            pl.BlockSpec((blk, D_IN), lambda i: (i, 0)),
            pl.BlockSpec((D_IN, D_HID), lambda i: (0, 0)),
            pl.BlockSpec((D_HID, D_HID), lambda i: (0, 0)),
            pl.BlockSpec((blk, D_AUG - D_HID), lambda i: (i, 0)),
        ],
        out_specs=pl.BlockSpec((blk, D_AUG), lambda i: (i, 0)),
        out_shape=jax.ShapeDtypeStruct((N_PAD, D_AUG), jnp.float32),
    )(x_pad, W_in, W_conv, aug)


def _combine_body(acc_ref, aug_ref, o_ref):
    ssum = acc_ref[0, :, :D_HID] + acc_ref[1, :, :D_HID]
    cnt = acc_ref[0, :, D_HID:D_HID + 1] + acc_ref[1, :, D_HID:D_HID + 1]
    inv = jnp.where(cnt > 0, 1.0 / cnt, 0.0)
    o_ref[...] = jnp.concatenate([ssum * inv, aug_ref[...]], axis=1)


def _combine(partials, aug):
    blk = 1024
    return pl.pallas_call(
        _combine_body,
        grid=(N_PAD // blk,),
        in_specs=[
            pl.BlockSpec((NC, blk, D_AUG), lambda i: (0, i, 0)),
            pl.BlockSpec((blk, D_AUG - D_HID), lambda i: (i, 0)),
        ],
        out_specs=pl.BlockSpec((blk, D_AUG), lambda i: (i, 0)),
        out_shape=jax.ShapeDtypeStruct((N_PAD, D_AUG), jnp.float32),
    )(partials, aug)


def _final_body(acc_ref, bconv_ref, wout_ref, bout_ref, o_ref):
    ssum = acc_ref[0, :, :D_HID] + acc_ref[1, :, :D_HID]
    cnt = acc_ref[0, :, D_HID:D_HID + 1] + acc_ref[1, :, D_HID:D_HID + 1]
    inv = jnp.where(cnt > 0, 1.0 / cnt, 0.0)
    h = jnp.maximum(ssum * inv + bconv_ref[...], 0.0)
    logits = jnp.dot(h, wout_ref[...],
                     preferred_element_type=jnp.float32) + bout_ref[...]
    m = jnp.max(logits, axis=1, keepdims=True)
    e = jnp.exp(logits - m)
    o_ref[...] = e / jnp.sum(e, axis=1, keepdims=True)


def _final(partials, b_conv2d, W_out, b_out2d):
    blk = 1000
    return pl.pallas_call(
        _final_body,
        grid=(N_NODES // blk,),
        in_specs=[
            pl.BlockSpec((NC, blk, D_AUG), lambda i: (0, i, 0)),
            pl.BlockSpec((1, D_HID), lambda i: (0, 0)),
            pl.BlockSpec((D_HID, D_OUT), lambda i: (0, 0)),
            pl.BlockSpec((1, D_OUT), lambda i: (0, 0)),
        ],
        out_specs=pl.BlockSpec((blk, D_OUT), lambda i: (i, 0)),
        out_shape=jax.ShapeDtypeStruct((N_NODES, D_OUT), jnp.float32),
    )(partials, b_conv2d, W_out, b_out2d)


def _sc_body(table, gidx, sidx, zrow, out_acc,
             gidx_v, sidx_v, rows_v, acc_sh, gsem0, gsem1, ssem0, ssem1):
    c = lax.axis_index("c")
    s = lax.axis_index("s")
    base = s * ROWS_PER_TILE
    gsems = (gsem0, gsem1)
    ssems = (ssem0, ssem1)
    ngroups = jnp.where(c == 0, G_SC0, G_SC1)
    gbase = jnp.where(c == 0, s * G_SC0, NS * G_SC0 + s * G_SC1)

    pltpu.sync_copy(zrow, acc_sh.at[pl.ds(base, ROWS_PER_TILE)])
    plsc.subcore_barrier()

    def group(g, carry):
        row0 = (gbase + g) * GRP
        pltpu.sync_copy(gidx.at[pl.ds(row0, GRP)], gidx_v)
        pltpu.sync_copy(sidx.at[pl.ds(row0, GRP)], sidx_v)

        pltpu.async_copy(table.at[gidx_v.at[0]], rows_v.at[0], gsems[0])
        for i in range(GRP):
            b = i & 1
            if i + 1 < GRP:
                pltpu.async_copy(table.at[gidx_v.at[i + 1]],
                                 rows_v.at[1 - b], gsems[1 - b])
            pltpu.make_async_copy(table.at[gidx_v.at[i]], rows_v.at[b],
                                  gsems[b]).wait()
            pltpu.sync_copy(rows_v.at[b], acc_sh.at[sidx_v.at[i]], add=True)
        return carry

    lax.fori_loop(0, ngroups, group, 0)
    plsc.subcore_barrier()

    pltpu.sync_copy(acc_sh.at[pl.ds(base, ROWS_PER_TILE)],
                    out_acc.at[c, pl.ds(base, ROWS_PER_TILE)])


_sc_phase = pl.kernel(
    _sc_body,
    out_type=[jax.ShapeDtypeStruct((NC, N_PAD, D_AUG), jnp.float32)],
    mesh=plsc.VectorSubcoreMesh(core_axis_name="c", subcore_axis_name="s"),
    scratch_types=[
        pltpu.VMEM((GRP, CHUNK), jnp.int32),
        pltpu.VMEM((GRP, CHUNK), jnp.int32),
        pltpu.VMEM((2, CHUNK, D_AUG), jnp.float32),
        pltpu.VMEM_SHARED((N_PAD, D_AUG), jnp.float32),
        pltpu.SemaphoreType.DMA,
        pltpu.SemaphoreType.DMA,
        pltpu.SemaphoreType.DMA,
        pltpu.SemaphoreType.DMA,
    ],
    compiler_params=pltpu.CompilerParams(use_tc_tiling_on_sc=False),
)


def kernel(x, hyperedge_index, W_in, W_conv, b_conv, W_out, b_out):
    x_pad = jnp.zeros((N_PAD, D_IN), jnp.float32).at[:N_NODES].set(x)
    aug = jnp.zeros((N_PAD, D_AUG - D_HID), jnp.float32).at[:N_NODES, 0].set(1.0)

    total = TOT_GRPS * GRP * CHUNK
    nchunks = TOT_GRPS * GRP
    npad = total - hyperedge_index.shape[1]
    pad = N_NODES + (jnp.arange(npad, dtype=jnp.int32) % (N_PAD - N_NODES))
    nid = jnp.concatenate([hyperedge_index[0], pad]).reshape(nchunks, CHUNK)
    hid = jnp.concatenate([hyperedge_index[1], pad]).reshape(nchunks, CHUNK)

    zrow = jnp.zeros((ROWS_PER_TILE, D_AUG), jnp.float32)

    xw = _input_matmul(x_pad, W_in, W_conv, aug)

    (s1p,) = _sc_phase(xw, nid, hid, zrow)
    he_aug = _combine(s1p, aug)

    (s2p,) = _sc_phase(he_aug, hid, nid, zrow)

    return _final(s2p, b_conv.reshape(1, D_HID), W_out,
                  b_out.reshape(1, D_OUT))


# speedup vs baseline: 1.0166x; 1.0166x over previous
import jax
import jax.numpy as jnp
from jax import lax
from jax.experimental import pallas as pl
from jax.experimental.pallas import tpu as pltpu
from jax.experimental.pallas import tpu_sc as plsc

N_NODES = 10000
D_IN = 128
D_HID = 128
D_OUT = 16

N_PAD = 10240
D_AUG = D_HID + 16
NC, NS = 2, 16
NT = NC * NS
CHUNK = 64
GRP = 16
TOT_GRPS = 320
G_SC0 = 10
G_SC1 = TOT_GRPS // NS - G_SC0
ROWS_PER_TILE = N_PAD // NS


def _mm2_body(x_ref, w1_ref, w2_ref, aug_ref, o_ref):
    h = jnp.maximum(jnp.dot(x_ref[...], w1_ref[...],
                            preferred_element_type=jnp.float32), 0.0)
    feat = jnp.dot(h, w2_ref[...], preferred_element_type=jnp.float32)
    o_ref[...] = jnp.concatenate([feat, aug_ref[...]], axis=1)


def _input_matmul(x_pad, W_in, W_conv, aug):
    blk = 1024
    return pl.pallas_call(
        _mm2_body,
        grid=(N_PAD // blk,),
        in_specs=[
            pl.BlockSpec((blk, D_IN), lambda i: (i, 0)),
            pl.BlockSpec((D_IN, D_HID), lambda i: (0, 0)),
            pl.BlockSpec((D_HID, D_HID), lambda i: (0, 0)),
            pl.BlockSpec((blk, D_AUG - D_HID), lambda i: (i, 0)),
        ],
        out_specs=pl.BlockSpec((blk, D_AUG), lambda i: (i, 0)),
        out_shape=jax.ShapeDtypeStruct((N_PAD, D_AUG), jnp.float32),
    )(x_pad, W_in, W_conv, aug)


def _combine_body(acc_ref, aug_ref, o_ref):
    ssum = acc_ref[0, :, :D_HID] + acc_ref[1, :, :D_HID]
    cnt = acc_ref[0, :, D_HID:D_HID + 1] + acc_ref[1, :, D_HID:D_HID + 1]
    inv = jnp.where(cnt > 0, 1.0 / cnt, 0.0)
    o_ref[...] = jnp.concatenate([ssum * inv, aug_ref[...]], axis=1)


def _combine(partials, aug):
    blk = 1024
    return pl.pallas_call(
        _combine_body,
        grid=(N_PAD // blk,),
        in_specs=[
            pl.BlockSpec((NC, blk, D_AUG), lambda i: (0, i, 0)),
            pl.BlockSpec((blk, D_AUG - D_HID), lambda i: (i, 0)),
        ],
        out_specs=pl.BlockSpec((blk, D_AUG), lambda i: (i, 0)),
        out_shape=jax.ShapeDtypeStruct((N_PAD, D_AUG), jnp.float32),
    )(partials, aug)


def _final_body(acc_ref, bconv_ref, wout_ref, bout_ref, o_ref):
    ssum = acc_ref[0, :, :D_HID] + acc_ref[1, :, :D_HID]
    cnt = acc_ref[0, :, D_HID:D_HID + 1] + acc_ref[1, :, D_HID:D_HID + 1]
    inv = jnp.where(cnt > 0, 1.0 / cnt, 0.0)
    h = jnp.maximum(ssum * inv + bconv_ref[...], 0.0)
    logits = jnp.dot(h, wout_ref[...],
                     preferred_element_type=jnp.float32) + bout_ref[...]
    m = jnp.max(logits, axis=1, keepdims=True)
    e = jnp.exp(logits - m)
    o_ref[...] = e / jnp.sum(e, axis=1, keepdims=True)


def _final(partials, b_conv2d, W_out, b_out2d):
    blk = 1000
    return pl.pallas_call(
        _final_body,
        grid=(N_NODES // blk,),
        in_specs=[
            pl.BlockSpec((NC, blk, D_AUG), lambda i: (0, i, 0)),
            pl.BlockSpec((1, D_HID), lambda i: (0, 0)),
            pl.BlockSpec((D_HID, D_OUT), lambda i: (0, 0)),
            pl.BlockSpec((1, D_OUT), lambda i: (0, 0)),
        ],
        out_specs=pl.BlockSpec((blk, D_OUT), lambda i: (i, 0)),
        out_shape=jax.ShapeDtypeStruct((N_NODES, D_OUT), jnp.float32),
    )(partials, b_conv2d, W_out, b_out2d)


def _sc_body(table, gidx, sidx, zrow, out_acc,
             gidx_v, sidx_v, rows_v, acc_sh, gsem0, gsem1, ssem0, ssem1):
    c = lax.axis_index("c")
    s = lax.axis_index("s")
    base = s * ROWS_PER_TILE
    gsems = (gsem0, gsem1)
    ssems = (ssem0, ssem1)
    ngroups = jnp.where(c == 0, G_SC0, G_SC1)
    gbase = jnp.where(c == 0, s * G_SC0, NS * G_SC0 + s * G_SC1)

    pltpu.sync_copy(zrow, acc_sh.at[pl.ds(base, ROWS_PER_TILE)])
    plsc.subcore_barrier()

    def group(g, carry):
        row0 = (gbase + g) * GRP
        pltpu.sync_copy(gidx.at[pl.ds(row0, GRP)], gidx_v)
        pltpu.sync_copy(sidx.at[pl.ds(row0, GRP)], sidx_v)

        pltpu.async_copy(table.at[gidx_v.at[0]], rows_v.at[0], gsems[0])
        for i in range(GRP):
            b = i & 1
            if i + 1 < GRP:
                pltpu.async_copy(table.at[gidx_v.at[i + 1]],
                                 rows_v.at[1 - b], gsems[1 - b])
            pltpu.make_async_copy(table.at[gidx_v.at[i]], rows_v.at[b],
                                  gsems[b]).wait()
            pltpu.sync_copy(rows_v.at[b], acc_sh.at[sidx_v.at[i]], add=True)
        return carry

    lax.fori_loop(0, ngroups, group, 0)
    plsc.subcore_barrier()

    pltpu.sync_copy(acc_sh.at[pl.ds(base, ROWS_PER_TILE)],
                    out_acc.at[c, pl.ds(base, ROWS_PER_TILE)])


_sc_phase = pl.kernel(
    _sc_body,
    out_type=[jax.ShapeDtypeStruct((NC, N_PAD, D_AUG), jnp.float32)],
    mesh=plsc.VectorSubcoreMesh(core_axis_name="c", subcore_axis_name="s"),
    scratch_types=[
        pltpu.VMEM((GRP, CHUNK), jnp.int32),
        pltpu.VMEM((GRP, CHUNK), jnp.int32),
        pltpu.VMEM((2, CHUNK, D_AUG), jnp.float32),
        pltpu.VMEM_SHARED((N_PAD, D_AUG), jnp.float32),
        pltpu.SemaphoreType.DMA,
        pltpu.SemaphoreType.DMA,
        pltpu.SemaphoreType.DMA,
        pltpu.SemaphoreType.DMA,
    ],
    compiler_params=pltpu.CompilerParams(use_tc_tiling_on_sc=False),
)


def kernel(x, hyperedge_index, W_in, W_conv, b_conv, W_out, b_out):
    x_pad = jnp.zeros((N_PAD, D_IN), jnp.float32).at[:N_NODES].set(x)
    aug = jnp.zeros((N_PAD, D_AUG - D_HID), jnp.float32).at[:N_NODES, 0].set(1.0)

    total = TOT_GRPS * GRP * CHUNK
    nchunks = TOT_GRPS * GRP
    npad = total - hyperedge_index.shape[1]
    pad = N_NODES + (jnp.arange(npad, dtype=jnp.int32) % (N_PAD - N_NODES))
    nid = jnp.concatenate([hyperedge_index[0], pad]).reshape(nchunks, CHUNK)
    hid = jnp.concatenate([hyperedge_index[1], pad]).reshape(nchunks, CHUNK)

    zrow = jnp.zeros((ROWS_PER_TILE, D_AUG), jnp.float32)

    xw = _input_matmul(x_pad, W_in, W_conv, aug)

    (s1p,) = _sc_phase(xw, nid, hid, zrow)
    he_aug = _combine(s1p, aug)

    (s2p,) = _sc_phase(he_aug, hid, nid, zrow)

    return _final(s2p, b_conv.reshape(1, D_HID), W_out,
                  b_out.reshape(1, D_OUT))
